# baseline (speedup 1.0000x reference)
import functools

import jax
import jax.numpy as jnp
from jax import lax
from jax.experimental import pallas as pl
from jax.experimental.pallas import tpu as pltpu

MARGIN_ = 0.02
BT_ = 256


def _row128(e):
    acc = e[:, 0:128]
    for c in range(1, e.shape[1] // 128):
        acc = acc + e[:, c * 128:(c + 1) * 128]
    return acc.astype(jnp.float32)


def _col8(e):
    h = e.shape[0]
    while h > 16:
        h //= 2
        e = e[:h, :] + e[h:2 * h, :]
    e = e.astype(jnp.float32)
    return e[:8, :] + e[8:16, :]


def _loss_kernel(n, nb, u_col, r_col, u_row, r_row, loss_ref, rowacc, colacc):
    ib = pl.program_id(0)

    @pl.when(ib == 0)
    def _init():
        rowacc[:, :] = jnp.zeros_like(rowacc)
        colacc[:, :] = jnp.zeros_like(colacc)

    rc = r_col[pl.ds(ib * BT_, BT_), :]
    mrc = (MARGIN_ - rc).astype(jnp.bfloat16)
    zero = jnp.bfloat16(0.0)

    rr_d = r_row[:, pl.ds(ib * BT_, BT_)].astype(jnp.bfloat16)
    e_d = jnp.maximum(mrc + rr_d, zero)
    ri = lax.broadcasted_iota(jnp.int32, (BT_, BT_), 0)
    ci = lax.broadcasted_iota(jnp.int32, (BT_, BT_), 1)
    e_d = jnp.where(ci > ri, e_d, zero)
    rowacc[pl.ds(ib * BT_, BT_), :] += _row128(e_d)
    colacc[:, pl.ds(ib * BT_, BT_)] += _col8(e_d)

    def body(jb, _):
        rr = r_row[:, pl.ds(jb * BT_, BT_)].astype(jnp.bfloat16)
        e = jnp.maximum(mrc + rr, zero)
        rowacc[pl.ds(ib * BT_, BT_), :] += _row128(e)
        colacc[:, pl.ds(jb * BT_, BT_)] += _col8(e)
        return 0

    lax.fori_loop(ib + 1, nb, body, 0)

    @pl.when(ib == nb - 1)
    def _final():
        num = jnp.sum(rowacc[:, :] * u_col[:, :]) - jnp.sum(
            colacc[:, :] * u_row[:, :])
        idx = lax.broadcasted_iota(jnp.int32, (1, n), 1)
        coef = ((n - 1) - 2 * idx).astype(jnp.float32)
        den = jnp.sum(u_row[:, :] * coef)
        loss_ref[0, 0] = num / (den + 1e-8)


@jax.jit
def kernel(pred, y_true):
    p = pred.reshape(-1).astype(jnp.float32)
    y = y_true.reshape(-1).astype(jnp.float32)
    n = p.shape[0]
    nb = n // BT_

    neg_u, r = lax.sort((-y, p), num_keys=1)
    u = -neg_u

    loss = pl.pallas_call(
        functools.partial(_loss_kernel, n, nb),
        grid=(nb,),
        in_specs=[
            pl.BlockSpec((n, 1), lambda i: (0, 0)),
            pl.BlockSpec((n, 1), lambda i: (0, 0)),
            pl.BlockSpec((1, n), lambda i: (0, 0)),
            pl.BlockSpec((1, n), lambda i: (0, 0)),
        ],
        out_specs=pl.BlockSpec(memory_space=pltpu.SMEM),
        out_shape=jax.ShapeDtypeStruct((1, 1), jnp.float32),
        scratch_shapes=[
            pltpu.VMEM((n, 128), jnp.float32),
            pltpu.VMEM((8, n), jnp.float32),
        ],
    )(u.reshape(n, 1), r.reshape(n, 1), u.reshape(1, n), r.reshape(1, n))

    return loss[0, 0]


# speedup vs baseline: 1.1029x; 1.1029x over previous
import functools

import jax
import jax.numpy as jnp
from jax import lax
from jax.experimental import pallas as pl
from jax.experimental.pallas import tpu as pltpu

MARGIN_ = 0.02
BT_ = 512


def _row128(e):
    acc = e[:, 0:128]
    for c in range(1, e.shape[1] // 128):
        acc = acc + e[:, c * 128:(c + 1) * 128]
    return acc.astype(jnp.float32)


def _col8(e):
    h = e.shape[0]
    while h > 16:
        h //= 2
        e = e[:h, :] + e[h:2 * h, :]
    e = e.astype(jnp.float32)
    return e[:8, :] + e[8:16, :]


def _loss_kernel(n, nb, u_col, r_col, u_row, r_row, loss_ref, rowacc, colacc):
    ib = pl.program_id(0)

    @pl.when(ib == 0)
    def _init():
        rowacc[:, :] = jnp.zeros_like(rowacc)
        colacc[:, :] = jnp.zeros_like(colacc)

    rc = r_col[pl.ds(ib * BT_, BT_), :]
    mrc = (MARGIN_ - rc).astype(jnp.bfloat16)
    zero = jnp.bfloat16(0.0)

    rr_d = r_row[:, pl.ds(ib * BT_, BT_)].astype(jnp.bfloat16)
    e_d = jnp.maximum(mrc + rr_d, zero)
    ri = lax.broadcasted_iota(jnp.int32, (BT_, BT_), 0)
    ci = lax.broadcasted_iota(jnp.int32, (BT_, BT_), 1)
    e_d = jnp.where(ci > ri, e_d, zero)
    rowacc[pl.ds(ib * BT_, BT_), :] += _row128(e_d)
    colacc[:, pl.ds(ib * BT_, BT_)] += _col8(e_d)

    def body(jb, _):
        rr = r_row[:, pl.ds(jb * BT_, BT_)].astype(jnp.bfloat16)
        e = jnp.maximum(mrc + rr, zero)
        rowacc[pl.ds(ib * BT_, BT_), :] += _row128(e)
        colacc[:, pl.ds(jb * BT_, BT_)] += _col8(e)
        return 0

    lax.fori_loop(ib + 1, nb, body, 0)

    @pl.when(ib == nb - 1)
    def _final():
        num = jnp.sum(rowacc[:, :] * u_col[:, :]) - jnp.sum(
            colacc[:, :] * u_row[:, :])
        idx = lax.broadcasted_iota(jnp.int32, (1, n), 1)
        coef = ((n - 1) - 2 * idx).astype(jnp.float32)
        den = jnp.sum(u_row[:, :] * coef)
        loss_ref[0, 0] = num / (den + 1e-8)


@jax.jit
def kernel(pred, y_true):
    p = pred.reshape(-1).astype(jnp.float32)
    y = y_true.reshape(-1).astype(jnp.float32)
    n = p.shape[0]
    nb = n // BT_

    neg_u, r = lax.sort((-y, p), num_keys=1)
    u = -neg_u

    loss = pl.pallas_call(
        functools.partial(_loss_kernel, n, nb),
        grid=(nb,),
        in_specs=[
            pl.BlockSpec((n, 1), lambda i: (0, 0)),
            pl.BlockSpec((n, 1), lambda i: (0, 0)),
            pl.BlockSpec((1, n), lambda i: (0, 0)),
            pl.BlockSpec((1, n), lambda i: (0, 0)),
        ],
        out_specs=pl.BlockSpec(memory_space=pltpu.SMEM),
        out_shape=jax.ShapeDtypeStruct((1, 1), jnp.float32),
        scratch_shapes=[
            pltpu.VMEM((n, 128), jnp.float32),
            pltpu.VMEM((8, n), jnp.float32),
        ],
    )(u.reshape(n, 1), r.reshape(n, 1), u.reshape(1, n), r.reshape(1, n))

    return loss[0, 0]
